# baseline (speedup 1.0000x reference)
import numpy as np

import jax
import jax.numpy as jnp
from jax import lax
from jax.experimental import pallas as pl
from jax.experimental.pallas import tpu as pltpu
from jax.experimental.pallas import tpu_sc as plsc

D = 128
K = 16
NC, NS = 2, 16
NW = NC * NS
SB = 8
IDX = SB * K
NBUF = 4
SPG = 8
GN = SB * SPG


CH = 112


def _pack_body(feats_hbm, out_hbm, b0, b1, o0, o1, si0, si1, so0, so1):
    bufs, outs = (b0, b1), (o0, o1)
    sis, sos = (si0, si1), (so0, so1)
    rpw = out_hbm.shape[0] // NW
    its = rpw // CH
    wid = lax.axis_index("s") * NC + lax.axis_index("c")
    rbase = wid * rpw

    def fire_in(t, p):
        tt = jnp.minimum(t, its - 1)
        pltpu.make_async_copy(feats_hbm.at[pl.ds(rbase + tt * CH, CH)],
                              bufs[p], sis[p]).start()

    def wait_in(p):
        pltpu.make_async_copy(feats_hbm.at[pl.ds(rbase, CH)],
                              bufs[p], sis[p]).wait()

    def fire_out(t, p):
        pltpu.make_async_copy(outs[p], out_hbm.at[pl.ds(rbase + t * CH, CH)],
                              sos[p]).start()

    def wait_out(p):
        pltpu.make_async_copy(outs[p], out_hbm.at[pl.ds(rbase, CH)],
                              sos[p]).wait()

    def compute(p):
        fb, ob = bufs[p], outs[p]

        def row(r, c0):
            for c2 in range(D // 32):
                a = fb[r, pl.ds(c2 * 32, 16)]
                b = fb[r, pl.ds(c2 * 32 + 16, 16)]
                pk = plsc.pack(a, b, format=plsc.PackFormat.INTERLEAVED)
                ob[r, pl.ds(c2 * 16, 16)] = plsc.bitcast(pk, jnp.int32)
            return c0
        lax.fori_loop(0, CH, row, 0, unroll=False)

    fire_in(0, 0)
    fire_in(1, 1)

    def step(t, carry):
        for u in range(2):
            tt = 2 * t + u
            wait_in(u)
            lax.cond(tt >= 2, lambda: wait_out(u), lambda: None)
            compute(u)
            fire_out(tt, u)
            fire_in(tt + 2, u)
        return carry
    lax.fori_loop(0, its // 2, step, 0, unroll=False)
    wait_in(0)
    wait_in(1)
    wait_out(0)
    wait_out(1)


def _make_pack(v_pad):
    mesh = plsc.VectorSubcoreMesh(core_axis_name="c", subcore_axis_name="s",
                                  num_cores=NC, num_subcores=NS)
    return pl.kernel(
        _pack_body,
        out_type=jax.ShapeDtypeStruct((v_pad, D // 2), jnp.int32),
        mesh=mesh,
        compiler_params=pltpu.CompilerParams(needs_layout_passes=False,
                                             use_tc_tiling_on_sc=False),
        scratch_types=[
            pltpu.VMEM((CH, D), jnp.float32),
            pltpu.VMEM((CH, D), jnp.float32),
            pltpu.VMEM((CH, D // 2), jnp.int32),
            pltpu.VMEM((CH, D // 2), jnp.int32),
            pltpu.SemaphoreType.DMA,
            pltpu.SemaphoreType.DMA,
            pltpu.SemaphoreType.DMA,
            pltpu.SemaphoreType.DMA,
        ],
    )


def _sc_gather_body(nodes_hbm, neig_hbm, feats_hbm, self_out, sum_out, *scr):
    sidx_v, nidx_v = scr[0], scr[1]
    nrows = scr[2:2 + NBUF]
    srows = scr[2 + NBUF:4 + NBUF]
    accs = scr[4 + NBUF:6 + NBUF]
    sgs = scr[6 + NBUF:6 + 2 * NBUF]
    sss = scr[6 + 2 * NBUF:8 + 2 * NBUF]
    sts = scr[8 + 2 * NBUF:10 + 2 * NBUF]
    bpw = self_out.shape[0] // NW
    steps = bpw // SB
    groups = bpw // GN
    last = steps - 1
    wid = lax.axis_index("s") * NC + lax.axis_index("c")
    base = wid * bpw

    pltpu.sync_copy(nodes_hbm.at[pl.ds(base, bpw)], sidx_v)
    pltpu.sync_copy(neig_hbm.at[pl.ds(base * K, bpw * K)], nidx_v)

    def fire_neigh(k, slot):
        kk = jnp.minimum(k, last)
        pltpu.make_async_copy(
            feats_hbm.at[nidx_v.at[pl.ds(kk * IDX, IDX)]],
            nrows[slot], sgs[slot]).start()

    def wait_neigh(slot):
        pltpu.make_async_copy(
            feats_hbm.at[nidx_v.at[pl.ds(0, IDX)]],
            nrows[slot], sgs[slot]).wait()

    def fire_self(g, p):
        pltpu.make_async_copy(
            feats_hbm.at[sidx_v.at[pl.ds(g * GN, GN)]],
            srows[p], sss[p]).start()

    def wait_self(p):
        pltpu.make_async_copy(
            feats_hbm.at[sidx_v.at[pl.ds(0, GN)]],
            srows[p], sss[p]).wait()

    def fire_store(g, p):
        gb = base + g * GN
        pltpu.make_async_copy(accs[p], sum_out.at[pl.ds(gb, GN)],
                              sts[p]).start()
        pltpu.make_async_copy(srows[p], self_out.at[pl.ds(gb, GN)],
                              sts[p]).start()

    def wait_store(p):
        pltpu.make_async_copy(accs[p], sum_out.at[pl.ds(0, GN)],
                              sts[p]).wait()
        pltpu.make_async_copy(srows[p], self_out.at[pl.ds(0, GN)],
                              sts[p]).wait()

    def compute_step(s, slot, p):
        nb = nrows[slot]
        ab = accs[p]
        hi_mask = jnp.int32(-65536)

        def node(n, c0):
            row = n * K
            outr = s * SB + n

            def chunk(c, c1):
                x = nb[row, pl.ds(c * 16, 16)]
                va = plsc.bitcast(x << 16, jnp.float32)
                vb = plsc.bitcast(x & hi_mask, jnp.float32)
                for j in range(1, K):
                    x = nb[row + j, pl.ds(c * 16, 16)]
                    va = va + plsc.bitcast(x << 16, jnp.float32)
                    vb = vb + plsc.bitcast(x & hi_mask, jnp.float32)
                ab[outr, pl.ds(c * 32, 16)] = va
                ab[outr, pl.ds(c * 32 + 16, 16)] = vb
                return c1
            return lax.fori_loop(0, D // 32, chunk, c0, unroll=False)
        lax.fori_loop(0, SB, node, 0, unroll=False)

    def run_group(g, p):
        def cycle(cyc, carry):
            for i in range(NBUF):
                wait_neigh(i)
                compute_step(cyc * NBUF + i, i, p)
                fire_neigh(g * SPG + cyc * NBUF + i + NBUF, i)
            return carry
        lax.fori_loop(0, SPG // NBUF, cycle, 0, unroll=False)
        wait_self(p)
        fire_store(g, p)

    for kk in range(NBUF):
        fire_neigh(kk, kk)
    fire_self(0, 0)
    fire_self(1, 1)
    for u in range(2):
        run_group(u, u)

    def pair_body(t, carry):
        for u in range(2):
            g = 2 * t + u
            wait_store(u)
            fire_self(g, u)
            run_group(g, u)
        return carry
    lax.fori_loop(1, groups // 2, pair_body, 0, unroll=False)

    for slot in range(NBUF):
        wait_neigh(slot)
    wait_store(0)
    wait_store(1)


def _make_sc_gather(b_pad):
    bpw = b_pad // NW
    mesh = plsc.VectorSubcoreMesh(core_axis_name="c", subcore_axis_name="s",
                                  num_cores=NC, num_subcores=NS)
    return pl.kernel(
        _sc_gather_body,
        out_type=[
            jax.ShapeDtypeStruct((b_pad, D // 2), jnp.int32),
            jax.ShapeDtypeStruct((b_pad, D), jnp.float32),
        ],
        mesh=mesh,
        compiler_params=pltpu.CompilerParams(needs_layout_passes=False,
                                             use_tc_tiling_on_sc=False),
        scratch_types=[
            pltpu.VMEM((bpw,), jnp.int32),
            pltpu.VMEM((bpw * K,), jnp.int32),
            *[pltpu.VMEM((IDX, D // 2), jnp.int32) for _ in range(NBUF)],
            pltpu.VMEM((GN, D // 2), jnp.int32),
            pltpu.VMEM((GN, D // 2), jnp.int32),
            pltpu.VMEM((GN, D), jnp.float32),
            pltpu.VMEM((GN, D), jnp.float32),
            *([pltpu.SemaphoreType.DMA] * NBUF),
            pltpu.SemaphoreType.DMA,
            pltpu.SemaphoreType.DMA,
            pltpu.SemaphoreType.DMA,
            pltpu.SemaphoreType.DMA,
        ],
    )


def _mm_body(self_ref, sum_ref, ws_ref, wn_ref, out_ref):
    acc = jnp.dot(self_ref[...], ws_ref[...],
                  preferred_element_type=jnp.float32)
    acc += jnp.dot(sum_ref[...], wn_ref[...] * (1.0 / K),
                   preferred_element_type=jnp.float32)
    out_ref[...] = jnp.maximum(acc, 0.0)


def _mm(self_feats, neigh_sum, w_self, w_neigh, bm):
    b = self_feats.shape[0]
    return pl.pallas_call(
        _mm_body,
        grid=(b // bm,),
        in_specs=[
            pl.BlockSpec((bm, D), lambda i: (i, 0)),
            pl.BlockSpec((bm, D), lambda i: (i, 0)),
            pl.BlockSpec((D, D), lambda i: (0, 0)),
            pl.BlockSpec((D, D), lambda i: (0, 0)),
        ],
        out_specs=pl.BlockSpec((bm, D), lambda i: (i, 0)),
        out_shape=jax.ShapeDtypeStruct((b, D), jnp.float32),
    )(self_feats, neigh_sum, w_self, w_neigh)


@jax.jit
def kernel(nodes, neig_nodes, feats, W_self, W_neigh):
    b = nodes.shape[0]
    bm = 2048
    b_pad = -(-b // bm) * bm
    nodes_p = jnp.zeros((b_pad,), jnp.int32).at[:b].set(nodes.astype(jnp.int32))
    neig_p = jnp.zeros((b_pad * K,), jnp.int32).at[:b * K].set(
        neig_nodes.reshape(-1).astype(jnp.int32))
    v = feats.shape[0]
    v_pad = -(-v // (NW * CH)) * (NW * CH)
    feats_p = jnp.zeros((v_pad, D), feats.dtype).at[:v].set(feats)
    feats_pk = _make_pack(v_pad)(feats_p)
    self_pk, neigh_sum = _make_sc_gather(b_pad)(nodes_p, neig_p, feats_pk)
    pos = np.arange(D)
    perm = (pos // 32) * 32 + ((pos % 32) // 2) + 16 * (pos % 2)
    self16 = lax.bitcast_convert_type(self_pk, jnp.bfloat16).reshape(b_pad, D)
    out = _mm(self16, neigh_sum, W_self[jnp.asarray(perm), :], W_neigh, bm=bm)
    return out[:b]


# speedup vs baseline: 1.0503x; 1.0503x over previous
import jax
import jax.numpy as jnp
from jax import lax
from jax.experimental import pallas as pl
from jax.experimental.pallas import tpu as pltpu
from jax.experimental.pallas import tpu_sc as plsc

D = 128
K = 16
NC, NS = 2, 16
NW = NC * NS
SB = 8
IDX = SB * K
NBUF = 4
SPG = 8
GN = SB * SPG


def _sc_gather_body(nodes_hbm, neig_hbm, feats_hbm, self_out, sum_out, *scr):
    sidx_v, nidx_v = scr[0], scr[1]
    nrows = scr[2:2 + NBUF]
    srows = scr[2 + NBUF:4 + NBUF]
    accs = scr[4 + NBUF:6 + NBUF]
    sgs = scr[6 + NBUF:6 + 2 * NBUF]
    sss = scr[6 + 2 * NBUF:8 + 2 * NBUF]
    sts = scr[8 + 2 * NBUF:10 + 2 * NBUF]
    bpw = self_out.shape[0] // NW
    steps = bpw // SB
    groups = bpw // GN
    last = steps - 1
    wid = lax.axis_index("s") * NC + lax.axis_index("c")
    base = wid * bpw

    pltpu.sync_copy(nodes_hbm.at[pl.ds(base, bpw)], sidx_v)
    pltpu.sync_copy(neig_hbm.at[pl.ds(base * K, bpw * K)], nidx_v)

    def fire_neigh(k, slot):
        kk = jnp.minimum(k, last)
        pltpu.make_async_copy(
            feats_hbm.at[nidx_v.at[pl.ds(kk * IDX, IDX)]],
            nrows[slot], sgs[slot]).start()

    def wait_neigh(slot):
        pltpu.make_async_copy(
            feats_hbm.at[nidx_v.at[pl.ds(0, IDX)]],
            nrows[slot], sgs[slot]).wait()

    def fire_self(g, p):
        pltpu.make_async_copy(
            feats_hbm.at[sidx_v.at[pl.ds(g * GN, GN)]],
            srows[p], sss[p]).start()

    def wait_self(p):
        pltpu.make_async_copy(
            feats_hbm.at[sidx_v.at[pl.ds(0, GN)]],
            srows[p], sss[p]).wait()

    def fire_store(g, p):
        gb = base + g * GN
        pltpu.make_async_copy(accs[p], sum_out.at[pl.ds(gb, GN)],
                              sts[p]).start()
        pltpu.make_async_copy(srows[p], self_out.at[pl.ds(gb, GN)],
                              sts[p]).start()

    def wait_store(p):
        pltpu.make_async_copy(accs[p], sum_out.at[pl.ds(0, GN)],
                              sts[p]).wait()
        pltpu.make_async_copy(srows[p], self_out.at[pl.ds(0, GN)],
                              sts[p]).wait()

    def compute_step(s, slot, p):
        nb = nrows[slot]
        ab = accs[p]

        def node(n, c0):
            row = n * K
            outr = s * SB + n

            def chunk(c, c1):
                v = nb[row, pl.ds(c * 16, 16)]
                for j in range(1, K):
                    v = v + nb[row + j, pl.ds(c * 16, 16)]
                ab[outr, pl.ds(c * 16, 16)] = v
                return c1
            return lax.fori_loop(0, D // 16, chunk, c0, unroll=False)
        lax.fori_loop(0, SB, node, 0, unroll=False)

    def run_group(g, p):
        def cycle(cyc, carry):
            for i in range(NBUF):
                wait_neigh(i)
                compute_step(cyc * NBUF + i, i, p)
                fire_neigh(g * SPG + cyc * NBUF + i + NBUF, i)
            return carry
        lax.fori_loop(0, SPG // NBUF, cycle, 0, unroll=False)
        wait_self(p)
        fire_store(g, p)

    for kk in range(NBUF):
        fire_neigh(kk, kk)
    fire_self(0, 0)
    fire_self(1, 1)
    for u in range(2):
        run_group(u, u)

    def pair_body(t, carry):
        for u in range(2):
            g = 2 * t + u
            wait_store(u)
            fire_self(g, u)
            run_group(g, u)
        return carry
    lax.fori_loop(1, groups // 2, pair_body, 0, unroll=False)

    for slot in range(NBUF):
        wait_neigh(slot)
    wait_store(0)
    wait_store(1)


def _make_sc_gather(b_pad):
    bpw = b_pad // NW
    mesh = plsc.VectorSubcoreMesh(core_axis_name="c", subcore_axis_name="s",
                                  num_cores=NC, num_subcores=NS)
    return pl.kernel(
        _sc_gather_body,
        out_type=[
            jax.ShapeDtypeStruct((b_pad, D), jnp.float32),
            jax.ShapeDtypeStruct((b_pad, D), jnp.float32),
        ],
        mesh=mesh,
        scratch_types=[
            pltpu.VMEM((bpw,), jnp.int32),
            pltpu.VMEM((bpw * K,), jnp.int32),
            *[pltpu.VMEM((IDX, D), jnp.float32) for _ in range(NBUF)],
            pltpu.VMEM((GN, D), jnp.float32),
            pltpu.VMEM((GN, D), jnp.float32),
            pltpu.VMEM((GN, D), jnp.float32),
            pltpu.VMEM((GN, D), jnp.float32),
            *([pltpu.SemaphoreType.DMA] * NBUF),
            pltpu.SemaphoreType.DMA,
            pltpu.SemaphoreType.DMA,
            pltpu.SemaphoreType.DMA,
            pltpu.SemaphoreType.DMA,
        ],
    )


def _mm_body(self_ref, sum_ref, ws_ref, wn_ref, out_ref):
    acc = jnp.dot(self_ref[...], ws_ref[...],
                  preferred_element_type=jnp.float32)
    acc += jnp.dot(sum_ref[...], wn_ref[...] * (1.0 / K),
                   preferred_element_type=jnp.float32)
    out_ref[...] = jnp.maximum(acc, 0.0)


def _mm(self_feats, neigh_sum, w_self, w_neigh, bm):
    b = self_feats.shape[0]
    return pl.pallas_call(
        _mm_body,
        grid=(b // bm,),
        in_specs=[
            pl.BlockSpec((bm, D), lambda i: (i, 0)),
            pl.BlockSpec((bm, D), lambda i: (i, 0)),
            pl.BlockSpec((D, D), lambda i: (0, 0)),
            pl.BlockSpec((D, D), lambda i: (0, 0)),
        ],
        out_specs=pl.BlockSpec((bm, D), lambda i: (i, 0)),
        out_shape=jax.ShapeDtypeStruct((b, D), jnp.float32),
    )(self_feats, neigh_sum, w_self, w_neigh)


@jax.jit
def kernel(nodes, neig_nodes, feats, W_self, W_neigh):
    b = nodes.shape[0]
    bm = 2048
    b_pad = -(-b // bm) * bm
    nodes_p = jnp.zeros((b_pad,), jnp.int32).at[:b].set(nodes.astype(jnp.int32))
    neig_p = jnp.zeros((b_pad * K,), jnp.int32).at[:b * K].set(
        neig_nodes.reshape(-1).astype(jnp.int32))
    self_feats, neigh_sum = _make_sc_gather(b_pad)(nodes_p, neig_p, feats)
    out = _mm(self_feats, neigh_sum, W_self, W_neigh, bm=bm)
    return out[:b]
